# baseline (speedup 1.0000x reference)
import jax
import jax.numpy as jnp
from jax import lax
from jax.experimental import pallas as pl
from jax.experimental.pallas import tpu as pltpu
from jax.experimental.pallas import tpu_sc as plsc

NC, NS = 2, 16
NW = NC * NS
B = 4096 * 200
D = 32
DP = 128
BPW = B // NW
CHUNK = 1600
NCHUNK = BPW // CHUNK


def _emb_body(idx_hbm, table_hbm, out_hbm, idx_v, staged, sem_g):
    wid = lax.axis_index("s") * NC + lax.axis_index("c")
    base = wid * BPW

    def step(i, c):
        off = base + i * CHUNK
        pltpu.sync_copy(idx_hbm.at[pl.ds(off, CHUNK)], idx_v)
        pltpu.async_copy(table_hbm.at[idx_v], staged, sem_g).wait()
        pltpu.sync_copy(staged, out_hbm.at[pl.ds(off, CHUNK), pl.ds(0, D)])
        return c

    lax.fori_loop(0, NCHUNK, step, 0)


@jax.jit
def _emb(ids_flat, weight):
    mesh = plsc.VectorSubcoreMesh(core_axis_name="c", subcore_axis_name="s",
                                  num_cores=NC, num_subcores=NS)
    return pl.kernel(
        _emb_body,
        out_type=jax.ShapeDtypeStruct((B, DP), jnp.float32),
        mesh=mesh,
        scratch_types=[
            pltpu.VMEM((CHUNK,), jnp.int32),
            pltpu.VMEM((CHUNK, D), jnp.float32),
            pltpu.SemaphoreType.DMA,
        ],
        compiler_params=pltpu.CompilerParams(use_tc_tiling_on_sc=False),
    )(ids_flat, weight)


def kernel(input_ids, weight):
    ids_flat = input_ids.reshape(-1).astype(jnp.int32)
    out = _emb(ids_flat, weight)
    out3 = out.reshape(input_ids.shape[0], input_ids.shape[1], DP)
    return out3[:, :, :D]


# speedup vs baseline: 1.0002x; 1.0002x over previous
import jax
import jax.numpy as jnp
from jax import lax
from jax.experimental import pallas as pl
from jax.experimental.pallas import tpu as pltpu
from jax.experimental.pallas import tpu_sc as plsc

NC, NS = 2, 16
NW = NC * NS
BATCH, HIST = 4096, 200
B = BATCH * HIST
D = 32
DP = 128
ROWS_PW = BATCH // NW
RCHUNK = 8
CHUNK = RCHUNK * HIST
NCHUNK = ROWS_PW // RCHUNK


def _emb_body(idx_hbm, table_hbm, out_hbm, idx_v, staged, sem_g):
    wid = lax.axis_index("s") * NC + lax.axis_index("c")
    row0 = wid * ROWS_PW

    def step(i, c):
        b0 = row0 + i * RCHUNK
        pltpu.sync_copy(idx_hbm.at[pl.ds(b0, RCHUNK)], idx_v)
        copies = [
            pltpu.async_copy(table_hbm.at[idx_v.at[j]],
                             staged.at[pl.ds(j * HIST, HIST)], sem_g)
            for j in range(RCHUNK)
        ]
        for cp in copies:
            cp.wait()
        off = b0 * HIST
        pltpu.sync_copy(staged, out_hbm.at[pl.ds(off, CHUNK), pl.ds(0, D)])
        return c

    lax.fori_loop(0, NCHUNK, step, 0)


@jax.jit
def _emb(ids, weight):
    mesh = plsc.VectorSubcoreMesh(core_axis_name="c", subcore_axis_name="s",
                                  num_cores=NC, num_subcores=NS)
    return pl.kernel(
        _emb_body,
        out_type=jax.ShapeDtypeStruct((B, DP), jnp.float32),
        mesh=mesh,
        scratch_types=[
            pltpu.VMEM((RCHUNK, HIST), jnp.int32),
            pltpu.VMEM((CHUNK, D), jnp.float32),
            pltpu.SemaphoreType.DMA,
        ],
        compiler_params=pltpu.CompilerParams(use_tc_tiling_on_sc=False),
    )(ids, weight)


def kernel(input_ids, weight):
    out = _emb(input_ids.astype(jnp.int32), weight)
    return out.reshape(BATCH, HIST, DP)[:, :, :D]


# speedup vs baseline: 1.0175x; 1.0172x over previous
import jax
import jax.numpy as jnp
from jax import lax
from jax.experimental import pallas as pl
from jax.experimental.pallas import tpu as pltpu
from jax.experimental.pallas import tpu_sc as plsc

NC, NS = 2, 16
NW = NC * NS
BATCH, HIST = 4096, 200
B = BATCH * HIST
D = 32
DP = 128
ROWS_PW = BATCH // NW
RCHUNK = 16
CHUNK = RCHUNK * HIST
NCHUNK = ROWS_PW // RCHUNK


def _emb_body(idx_hbm, table_hbm, out_hbm, idx_v, staged, sem_g):
    wid = lax.axis_index("s") * NC + lax.axis_index("c")
    row0 = wid * ROWS_PW

    def step(i, c):
        b0 = row0 + i * RCHUNK
        pltpu.sync_copy(idx_hbm.at[pl.ds(b0, RCHUNK)], idx_v)
        copies = [
            pltpu.async_copy(table_hbm.at[idx_v.at[j]],
                             staged.at[pl.ds(j * HIST, HIST)], sem_g)
            for j in range(RCHUNK)
        ]
        for cp in copies:
            cp.wait()
        off = b0 * HIST
        pltpu.sync_copy(staged, out_hbm.at[pl.ds(off, CHUNK), pl.ds(0, D)])
        return c

    lax.fori_loop(0, NCHUNK, step, 0)


@jax.jit
def _emb(ids, weight):
    mesh = plsc.VectorSubcoreMesh(core_axis_name="c", subcore_axis_name="s",
                                  num_cores=NC, num_subcores=NS)
    return pl.kernel(
        _emb_body,
        out_type=jax.ShapeDtypeStruct((B, DP), jnp.float32),
        mesh=mesh,
        scratch_types=[
            pltpu.VMEM((RCHUNK, HIST), jnp.int32),
            pltpu.VMEM((CHUNK, D), jnp.float32),
            pltpu.SemaphoreType.DMA,
        ],
        compiler_params=pltpu.CompilerParams(use_tc_tiling_on_sc=False),
    )(ids, weight)


def kernel(input_ids, weight):
    out = _emb(input_ids.astype(jnp.int32), weight)
    return out.reshape(BATCH, HIST, DP)[:, :, :D]
